# Initial kernel scaffold; baseline (speedup 1.0000x reference)
import jax
import jax.numpy as jnp
from jax.experimental import pallas as pl


def kernel(x, edge_index, batch, W1, b1, W2, b2, W3, b3, W4, b4, Wc1, bc1, gamma, beta, Wc2, bc2, Wc3, bc3, Wc4, bc4):
    raise NotImplementedError("write your pallas kernel here")


# speedup vs baseline: 6.5803x; 6.5803x over previous
import functools
import jax
import jax.numpy as jnp
from jax import lax
from jax.experimental import pallas as pl
from jax.experimental.pallas import tpu as pltpu
from jax.experimental.pallas import tpu_sc as plsc

NC, NS = 2, 16
NW = NC * NS
N_RAW = 10000
N = 10240
HALF = N // 2
HPT = HALF // NS
E = 320000
EPW = E // NW
EPT = E // NS
CH = 80
D_HID = 128
D_SMALL = 16
KP = 16
B = 64
SEL = KP * B
SELPW = SEL // NW
D_CAT = 512

_f32 = jnp.float32


@functools.lru_cache(maxsize=None)
def _get_mesh():
  return plsc.VectorSubcoreMesh(core_axis_name="c", subcore_axis_name="s",
                                num_cores=NC, num_subcores=NS)


@functools.lru_cache(maxsize=None)
def _make_agg():
  nch = EPT // CH

  @functools.partial(
      pl.kernel,
      out_type=jax.ShapeDtypeStruct((NC, HALF, D_HID), _f32),
      mesh=_get_mesh(),
      scratch_types=[
          pltpu.VMEM((CH,), jnp.int32),
          pltpu.VMEM((CH,), jnp.int32),
          pltpu.VMEM((CH, D_HID), _f32),
          pltpu.VMEM((HPT, D_HID), _f32),
          pltpu.VMEM_SHARED((HALF + 8, D_HID), _f32),
          pltpu.SemaphoreType.DMA,
      ],
  )
  def k(y_hbm, src_hbm, dst_hbm, zeros_hbm, out_hbm,
        sidx, didx, rows, bounce, acc, sem):
    c = lax.axis_index("c")
    s = lax.axis_index("s")
    base_c = (c * HALF).astype(jnp.int32)
    pltpu.sync_copy(zeros_hbm, bounce)
    pltpu.sync_copy(bounce, acc.at[pl.ds(s * HPT, HPT), :])
    plsc.subcore_barrier()
    ebase = s * EPT

    def body(i, carry):
      off = ebase + i * CH
      pltpu.sync_copy(src_hbm.at[pl.ds(off, CH)], sidx)
      pltpu.sync_copy(dst_hbm.at[pl.ds(off, CH)], didx)
      for j in range(CH // 16):
        d = didx[pl.ds(j * 16, 16)]
        loc = d - base_c
        oob = (loc < 0) | (loc >= HALF)
        didx[pl.ds(j * 16, 16)] = jnp.where(oob, HALF, loc)
      pltpu.async_copy(y_hbm.at[sidx], rows, sem).wait()
      pltpu.sync_copy(rows, acc.at[didx], add=True)
      return carry

    lax.fori_loop(0, nch, body, 0)
    plsc.subcore_barrier()
    pltpu.sync_copy(acc.at[pl.ds(s * HPT, HPT), :], bounce)
    pltpu.sync_copy(bounce, out_hbm.at[c, pl.ds(s * HPT, HPT), :])

  return k


@functools.lru_cache(maxsize=None)
def _make_agg_y4():
  nch = EPW // CH

  @functools.partial(
      pl.kernel,
      out_type=jax.ShapeDtypeStruct((NW, N), _f32),
      mesh=_get_mesh(),
      compiler_params=pltpu.CompilerParams(needs_layout_passes=False),
      scratch_types=[
          pltpu.VMEM((CH,), jnp.int32),
          pltpu.VMEM((CH,), jnp.int32),
          pltpu.VMEM((N,), _f32),
          pltpu.VMEM((N,), _f32),
      ],
  )
  def k(y4_hbm, src_hbm, dst_hbm, out_hbm, sidx, didx, y4_v, acc_v):
    c = lax.axis_index("c")
    s = lax.axis_index("s")
    w = s * NC + c
    pltpu.sync_copy(y4_hbm, y4_v)
    zero = jnp.zeros((16,), _f32)

    def zbody(i, carry):
      acc_v[pl.ds(i * 16, 16)] = zero
      return carry

    lax.fori_loop(0, N // 16, zbody, 0)
    base = w * EPW

    def body(i, carry):
      off = base + i * CH
      pltpu.sync_copy(src_hbm.at[pl.ds(off, CH)], sidx)
      pltpu.sync_copy(dst_hbm.at[pl.ds(off, CH)], didx)
      for j in range(CH // 16):
        sv = sidx[pl.ds(j * 16, 16)]
        dv = didx[pl.ds(j * 16, 16)]
        vals = plsc.load_gather(y4_v, [sv])
        plsc.addupdate_scatter(acc_v, [dv], vals)
      return carry

    lax.fori_loop(0, nch, body, 0)
    pltpu.sync_copy(acc_v, out_hbm.at[w, :])

  return k


@functools.lru_cache(maxsize=None)
def _make_gather():
  @functools.partial(
      pl.kernel,
      out_type=jax.ShapeDtypeStruct((SEL, D_CAT), _f32),
      mesh=_get_mesh(),
      scratch_types=[
          pltpu.VMEM((SELPW,), jnp.int32),
          pltpu.VMEM((SELPW, D_CAT), _f32),
          pltpu.SemaphoreType.DMA,
      ],
  )
  def k(table_hbm, idx_hbm, out_hbm, idx_v, rows_v, sem):
    c = lax.axis_index("c")
    s = lax.axis_index("s")
    w = s * NC + c
    base = w * SELPW
    pltpu.sync_copy(idx_hbm.at[pl.ds(base, SELPW)], idx_v)
    pltpu.async_copy(table_hbm.at[idx_v], rows_v, sem).wait()
    pltpu.sync_copy(rows_v, out_hbm.at[pl.ds(base, SELPW), :])

  return k


def _dot(a, b):
  return jax.lax.dot_general(a, b, (((1,), (0,)), ((), ())),
                             preferred_element_type=_f32)


def _tc_call(body, out_shapes):
  return pl.pallas_call(body, out_shape=out_shapes)


def _tc0_body(degp_ref, x_ref, w1_ref, dinv_ref, y1_ref):
  deg = jnp.sum(degp_ref[...], axis=0)[:, None] + 1.0
  row = lax.broadcasted_iota(jnp.int32, (N, 1), 0)
  dinv = jnp.where(row < N_RAW, 1.0 / jnp.sqrt(deg), 0.0)
  dinv_ref[...] = dinv
  y1_ref[...] = _dot(x_ref[...], w1_ref[...]) * dinv


def _tc_layer_body(accp_ref, y_ref, dinv_ref, b_ref, wn_ref, h_ref, yn_ref):
  dinv = dinv_ref[...]
  agg = jnp.concatenate([accp_ref[0], accp_ref[1]], axis=0) + y_ref[...]
  h = jnp.tanh(agg * dinv + b_ref[...][None, :])
  h_ref[...] = h
  yn_ref[...] = _dot(h, wn_ref[...]) * dinv


def _tc3_body(accp_ref, y_ref, dinv_ref, b_ref, w4_ref, h_ref, y4_ref):
  dinv = dinv_ref[...]
  agg = jnp.concatenate([accp_ref[0], accp_ref[1]], axis=0) + y_ref[...]
  h = jnp.tanh(agg * dinv + b_ref[...][None, :])
  h_ref[...] = h
  y4_ref[...] = _dot(h, w4_ref[...]) * dinv


def _tc4a_body(accp_ref, y4_ref, dinv_ref, b4_ref, h1_ref, h2_ref, h3_ref,
               h4_ref, hcat_ref):
  agg = jnp.sum(accp_ref[...], axis=0)[:, None] + y4_ref[...]
  h4 = jnp.tanh(agg * dinv_ref[...] + b4_ref[...][None, :])
  h4_ref[...] = h4
  hcat_ref[...] = jnp.concatenate(
      [h1_ref[...], h2_ref[...], h3_ref[...], h4,
       jnp.zeros((N, D_CAT - 3 * D_HID - 1), _f32)], axis=1)


_BIG = 2**30


def _tc4b_body(u_ref, batch_ref, sel_ref, val_ref):
  batch2 = batch_ref[...]
  giota = lax.broadcasted_iota(jnp.int32, (B, N // 128, 128), 0)
  member = batch2[None, :, :] == giota
  flat = lax.broadcasted_iota(jnp.int32, (N // 128, 128), 0) * 128 + \
      lax.broadcasted_iota(jnp.int32, (N // 128, 128), 1)

  def step(t, carry):
    u, s_acc, v_acc = carry
    masked = jnp.where(member, u[None, :, :], -3.0)
    segmax = jnp.max(masked, axis=(1, 2))
    valid = segmax > -2.5
    ismax = member & (u[None, :, :] == segmax[:, None, None])
    cand = jnp.where(ismax, flat[None, :, :], _BIG)
    segidx = jnp.min(cand, axis=(1, 2))
    safe_idx = jnp.where(valid, segidx, 0)
    chosen = (flat[None, :, :] == segidx[:, None, None]) & valid[:, None, None]
    kill = jnp.any(chosen, axis=0)
    u = jnp.where(kill, -3.0, u)
    rowmask = lax.broadcasted_iota(jnp.int32, (KP, B), 0) == t
    s_acc = jnp.where(rowmask, safe_idx[None, :], s_acc)
    v_acc = jnp.where(rowmask, jnp.where(valid, 1.0, 0.0)[None, :], v_acc)
    return u, s_acc, v_acc

  u0 = u_ref[...]
  s0 = jnp.zeros((KP, B), jnp.int32)
  v0 = jnp.zeros((KP, B), _f32)
  _, s_fin, v_fin = lax.fori_loop(0, 15, step, (u0, s0, v0))
  sel_ref[...] = s_fin
  val_ref[...] = v_fin


def _tc5_body(p_ref, val_ref, wc1_ref, bc1_ref, gamma_ref, beta_ref,
              wc2_ref, bc2_ref, wc3_ref, bc3_ref, wc4_ref, bc4_ref, out_ref):
  p = p_ref[...].reshape(KP, B, D_CAT)
  val = val_ref[...]
  z = jnp.zeros((B, 256), _f32)
  for t in range(15):
    pt = p[t, :, :385] * val[t][:, None]
    z = z + _dot(pt, wc1_ref[t])
  z = z + bc1_ref[...][None, :]
  z = (z / jnp.sqrt(1.0 + 1e-5)) * gamma_ref[...][None, :] + \
      beta_ref[...][None, :]
  z = jnp.maximum(z, 0.0)
  z = jnp.maximum(_dot(z, wc2_ref[...]) + bc2_ref[...][None, :], 0.0)
  z = jnp.maximum(_dot(z, wc3_ref[...]) + bc3_ref[...][None, :], 0.0)
  out_ref[...] = _dot(z, wc4_ref[...]) + bc4_ref[...][None, :]


def kernel(x, edge_index, batch, W1, b1, W2, b2, W3, b3, W4, b4,
           Wc1, bc1, gamma, beta, Wc2, bc2, Wc3, bc3, Wc4, bc4):
  x_pad = jnp.pad(x, ((0, N - N_RAW), (0, 0)))
  src = edge_index[0]
  dst = edge_index[1]
  batch_pad = jnp.pad(batch, (0, N - N_RAW), constant_values=B)
  batch2d = batch_pad.reshape(N // 128, 128)

  zeros_h = jnp.zeros((HPT, D_HID), _f32)
  ones_n = jnp.ones((N,), _f32)

  degp = _make_agg_y4()(ones_n, dst, dst)

  dinv, y1 = _tc_call(_tc0_body, [
      jax.ShapeDtypeStruct((N, 1), _f32),
      jax.ShapeDtypeStruct((N, D_HID), _f32),
  ])(degp, x_pad, W1)

  agg = _make_agg()

  layer_out = [
      jax.ShapeDtypeStruct((N, D_HID), _f32),
      jax.ShapeDtypeStruct((N, D_HID), _f32),
  ]
  acc1 = agg(y1, src, dst, zeros_h)
  h1, y2 = _tc_call(_tc_layer_body, layer_out)(acc1, y1, dinv, b1, W2)
  acc2 = agg(y2, src, dst, zeros_h)
  h2, y3 = _tc_call(_tc_layer_body, layer_out)(acc2, y2, dinv, b2, W3)
  acc3 = agg(y3, src, dst, zeros_h)
  h3, y4 = _tc_call(_tc3_body, [
      jax.ShapeDtypeStruct((N, D_HID), _f32),
      jax.ShapeDtypeStruct((N, 1), _f32),
  ])(acc3, y3, dinv, b3, W4)

  acc4 = _make_agg_y4()(y4.reshape(N), src, dst)

  h4, hcat = _tc_call(_tc4a_body, [
      jax.ShapeDtypeStruct((N, 1), _f32),
      jax.ShapeDtypeStruct((N, D_CAT), _f32),
  ])(acc4, y4, dinv, b4, h1, h2, h3)

  u2d = h4.reshape(N // 128, 128)
  sel, val = _tc_call(_tc4b_body, [
      jax.ShapeDtypeStruct((KP, B), jnp.int32),
      jax.ShapeDtypeStruct((KP, B), _f32),
  ])(u2d, batch2d)

  pooled = _make_gather()(hcat, sel.reshape(SEL))

  Wc1v = Wc1.reshape(15, 385, 256)
  out = _tc_call(_tc5_body, jax.ShapeDtypeStruct((B, 1), _f32))(
      pooled, val, Wc1v, bc1, gamma, beta, Wc2, bc2, Wc3, bc3, Wc4, bc4)
  return out
